# baseline (speedup 1.0000x reference)
import functools

import jax
import jax.numpy as jnp
from jax import lax
from jax.experimental import pallas as pl
from jax.experimental.pallas import tpu as pltpu
from jax.experimental.pallas import tpu_sc as plsc

N_NODES = 10000
N_EDGES = 320000
D_IN = 128
D_HID = 128
D_OUT = 64

N_PAD = 10240
NUM_TILES = 32
E_PER_TILE = 10240
E_PAD = E_PER_TILE * NUM_TILES


def _allreduce_sum16(v):
    lane = lax.iota(jnp.int32, 16)
    for sh in (8, 4, 2, 1):
        v = v + jnp.take_along_axis(v, jnp.bitwise_xor(lane, sh), axis=0)
    return v


def _make_edge_kernel(d, interpret=False):
    chunk = 64 if d >= 128 else 128
    k_sub = d // 16
    rows_per_tile = N_PAD // 16
    n_chunks = E_PER_TILE // chunk
    mesh = plsc.VectorSubcoreMesh(
        core_axis_name="c", subcore_axis_name="s", num_cores=2, num_subcores=16
    )

    @functools.partial(
        pl.kernel,
        out_type=(jax.ShapeDtypeStruct((2, N_PAD, d), jnp.float32),
                  jax.ShapeDtypeStruct((NUM_TILES, N_PAD), jnp.float32)),
        mesh=mesh,
        scratch_types=[
            pltpu.VMEM((2, chunk), jnp.int32),
            pltpu.VMEM((2, chunk), jnp.int32),
            pltpu.VMEM((chunk, d), jnp.float32),
            pltpu.VMEM((chunk, d), jnp.float32),
            pltpu.VMEM((chunk, d), jnp.float32),
            pltpu.VMEM((chunk, d), jnp.float32),
            pltpu.VMEM((d,), jnp.float32),
            pltpu.VMEM((N_PAD,), jnp.float32),
            pltpu.VMEM((128,), jnp.float32),
            pltpu.VMEM_SHARED((N_PAD, d), jnp.float32),
            pltpu.SemaphoreType.DMA,
            pltpu.SemaphoreType.DMA,
        ],
        compiler_params=pltpu.CompilerParams(use_tc_tiling_on_sc=False,
                                             needs_layout_passes=False),
        interpret=interpret,
    )
    def edge_kernel(xl_hbm, xr_hbm, att_hbm, idx_hbm, zeros_hbm,
                    out_hbm, den_hbm, sidx0, sidx1, arows0,
                    brows0, arows1, brows1, attv, denv, exbuf, acc_sp,
                    sem0, sem1):
        cid = lax.axis_index("c")
        sid = lax.axis_index("s")
        wid = sid * 2 + cid
        row0 = sid * rows_per_tile

        pltpu.sync_copy(zeros_hbm.at[pl.ds(row0, rows_per_tile)],
                        acc_sp.at[pl.ds(row0, rows_per_tile)])
        pltpu.sync_copy(att_hbm, attv)
        zv = jnp.zeros((16,), jnp.float32)

        def zero_body(i, carry):
            denv[pl.ds(i * 16, 16)] = zv
            return carry

        lax.fori_loop(0, N_PAD // 16, zero_body, 0)
        plsc.subcore_barrier()

        att_vs = [attv[pl.ds(k * 16, 16)] for k in range(k_sub)]
        lane = lax.iota(jnp.int32, 16)
        lowbit = jnp.bitwise_and(lane, 1)
        oddlane = lowbit == 1
        lane01 = lane < 2
        cbase = wid * n_chunks
        bufs = [(sidx0, arows0, brows0, sem0),
                (sidx1, arows1, brows1, sem1)]

        def start_gathers(ci, buf):
            sidx, arows, brows, sem = buf
            pltpu.sync_copy(idx_hbm.at[cbase + ci], sidx)
            pltpu.async_copy(xl_hbm.at[sidx.at[0]], arows, sem)
            pltpu.async_copy(xr_hbm.at[sidx.at[1]], brows, sem)

        def wait_gathers(buf):
            sidx, arows, brows, sem = buf
            pltpu.make_async_copy(xl_hbm.at[sidx.at[0]], arows, sem).wait()
            pltpu.make_async_copy(xr_hbm.at[sidx.at[1]], brows, sem).wait()

        start_gathers(0, bufs[0])

        def outer_body(co, carry):
            for b in (0, 1):
                ci = 2 * co + b
                sidx, arows, brows, sem = bufs[b]
                dstv = sidx.at[1]
                wait_gathers(bufs[b])
                start_gathers(jnp.minimum(ci + 1, n_chunks - 1), bufs[1 - b])

                @plsc.parallel_loop(0, chunk // 2, unroll=3)
                def edge_loop(jj):
                    j0 = 2 * jj
                    j1 = j0 + 1
                    avs0, avs1 = [], []
                    s0 = jnp.zeros((16,), jnp.float32)
                    s1 = jnp.zeros((16,), jnp.float32)
                    for k in range(k_sub):
                        a0 = arows[j0, pl.ds(k * 16, 16)]
                        b0 = brows[j0, pl.ds(k * 16, 16)]
                        a1 = arows[j1, pl.ds(k * 16, 16)]
                        b1 = brows[j1, pl.ds(k * 16, 16)]
                        z0 = a0 + b0
                        z1 = a1 + b1
                        s0 = s0 + jnp.maximum(z0, 0.2 * z0) * att_vs[k]
                        s1 = s1 + jnp.maximum(z1, 0.2 * z1) * att_vs[k]
                        avs0.append(a0)
                        avs1.append(a1)
                    ex0 = jnp.exp(_allreduce_sum16(s0))
                    ex1 = jnp.exp(_allreduce_sum16(s1))
                    for k in range(k_sub):
                        arows[j0, pl.ds(k * 16, 16)] = avs0[k] * ex0
                        arows[j1, pl.ds(k * 16, 16)] = avs1[k] * ex1
                    exmine = jnp.where(oddlane, ex1, ex0)
                    plsc.store_scatter(exbuf, [j0 + lowbit], exmine,
                                       mask=lane01)

                def den_sweep(g, carry2):
                    djs = sidx[1, pl.ds(g * 16, 16)]
                    exs = exbuf[pl.ds(g * 16, 16)]
                    old = plsc.load_gather(denv, [djs])
                    tot = exs
                    for sh in range(1, 16):
                        perm = jnp.bitwise_xor(lane, sh)
                        dperm = jnp.take_along_axis(djs, perm, axis=0)
                        eperm = jnp.take_along_axis(exs, perm, axis=0)
                        tot = tot + jnp.where(djs == dperm, eperm, 0.0)
                    plsc.store_scatter(denv, [djs], old + tot)
                    return carry2

                lax.fori_loop(0, chunk // 16, den_sweep, 0)
                pltpu.sync_copy(arows, acc_sp.at[dstv], add=True)
            return carry

        lax.fori_loop(0, n_chunks // 2, outer_body, 0)
        wait_gathers(bufs[0])
        pltpu.sync_copy(denv, den_hbm.at[wid])
        plsc.subcore_barrier()
        pltpu.sync_copy(acc_sp.at[pl.ds(row0, rows_per_tile)],
                        out_hbm.at[cid, pl.ds(row0, rows_per_tile)])

    return edge_kernel


_ROWS = 640
_GRID = N_PAD // _ROWS


def _mm3(x, w1, w2, w3, interpret=False):
    dm = w1.shape[1]

    def body(x_ref, w1_ref, w2_ref, w3_ref, o1_ref, o2_ref, o3_ref):
        xb = x_ref[...]
        o1_ref[...] = jnp.dot(xb, w1_ref[...], preferred_element_type=jnp.float32)
        o2_ref[...] = jnp.dot(xb, w2_ref[...], preferred_element_type=jnp.float32)
        o3_ref[...] = jnp.dot(xb, w3_ref[...], preferred_element_type=jnp.float32)

    k = x.shape[1]
    return pl.pallas_call(
        body,
        grid=(_GRID,),
        in_specs=[
            pl.BlockSpec((_ROWS, k), lambda i: (i, 0)),
            pl.BlockSpec((k, dm), lambda i: (0, 0)),
            pl.BlockSpec((k, dm), lambda i: (0, 0)),
            pl.BlockSpec((k, dm), lambda i: (0, 0)),
        ],
        out_specs=[pl.BlockSpec((_ROWS, dm), lambda i: (i, 0))] * 3,
        out_shape=[jax.ShapeDtypeStruct((N_PAD, dm), jnp.float32)] * 3,
        interpret=interpret,
    )(x, w1, w2, w3)


def _mid(acc, den, lin1, b1, l1b, wl2, wr2, l2w, interpret=False):
    d = lin1.shape[1]
    dm = wl2.shape[1]

    def body(acc_ref, den_ref, lin_ref, b1_ref, l1b_ref, wl_ref, wr_ref,
             lw_ref, o1_ref, o2_ref, o3_ref):
        num = acc_ref[0] + acc_ref[1]
        den_b = jnp.sum(den_ref[...], axis=0)[:, None]
        h = num / (den_b + 1e-16) + lin_ref[...] + b1_ref[...] + l1b_ref[...]
        h = jnp.maximum(h, 0.0)
        o1_ref[...] = jnp.dot(h, wl_ref[...], preferred_element_type=jnp.float32)
        o2_ref[...] = jnp.dot(h, wr_ref[...], preferred_element_type=jnp.float32)
        o3_ref[...] = jnp.dot(h, lw_ref[...], preferred_element_type=jnp.float32)

    return pl.pallas_call(
        body,
        grid=(_GRID,),
        in_specs=[
            pl.BlockSpec((2, _ROWS, d), lambda i: (0, i, 0)),
            pl.BlockSpec((NUM_TILES, _ROWS), lambda i: (0, i)),
            pl.BlockSpec((_ROWS, d), lambda i: (i, 0)),
            pl.BlockSpec((1, d), lambda i: (0, 0)),
            pl.BlockSpec((1, d), lambda i: (0, 0)),
            pl.BlockSpec((d, dm), lambda i: (0, 0)),
            pl.BlockSpec((d, dm), lambda i: (0, 0)),
            pl.BlockSpec((d, dm), lambda i: (0, 0)),
        ],
        out_specs=[pl.BlockSpec((_ROWS, dm), lambda i: (i, 0))] * 3,
        out_shape=[jax.ShapeDtypeStruct((N_PAD, dm), jnp.float32)] * 3,
        interpret=interpret,
    )(acc, den, lin1, b1, l1b, wl2, wr2, l2w)


def _final(acc, den, lin2, b2, l2b, interpret=False):
    d = lin2.shape[1]

    def body(acc_ref, den_ref, lin_ref, b2_ref, l2b_ref, o_ref):
        num = acc_ref[0] + acc_ref[1]
        den_b = jnp.sum(den_ref[...], axis=0)[:, None]
        o = num / (den_b + 1e-16) + lin_ref[...] + b2_ref[...] + l2b_ref[...]
        m = jnp.max(o, axis=1, keepdims=True)
        ex = jnp.exp(o - m)
        lse = m + jnp.log(jnp.sum(ex, axis=1, keepdims=True))
        o_ref[...] = o - lse

    return pl.pallas_call(
        body,
        grid=(_GRID,),
        in_specs=[
            pl.BlockSpec((2, _ROWS, d), lambda i: (0, i, 0)),
            pl.BlockSpec((NUM_TILES, _ROWS), lambda i: (0, i)),
            pl.BlockSpec((_ROWS, d), lambda i: (i, 0)),
            pl.BlockSpec((1, d), lambda i: (0, 0)),
            pl.BlockSpec((1, d), lambda i: (0, 0)),
        ],
        out_specs=pl.BlockSpec((_ROWS, d), lambda i: (i, 0)),
        out_shape=jax.ShapeDtypeStruct((N_PAD, d), jnp.float32),
        interpret=interpret,
    )(acc, den, lin2, b2, l2b)


def kernel(x, edge_index, Wl1, Wr1, att1, b1, L1w, L1b, Wl2, Wr2, att2, b2,
           L2w, L2b):
    src = edge_index[0].astype(jnp.int32)
    dst = edge_index[1].astype(jnp.int32)
    per_tile_pad = E_PER_TILE - N_EDGES // NUM_TILES
    pad_rows = N_NODES + jnp.arange(per_tile_pad, dtype=jnp.int32)
    pad_blk = jnp.broadcast_to(pad_rows, (NUM_TILES, per_tile_pad))

    def _pad_edges(v):
        v = v.reshape(NUM_TILES, N_EDGES // NUM_TILES)
        return jnp.concatenate([v, pad_blk], axis=1).reshape(-1)

    src_p = _pad_edges(src)
    dst_p = _pad_edges(dst)

    def _pack_idx(chunk):
        s2 = src_p.reshape(-1, chunk)
        d2 = dst_p.reshape(-1, chunk)
        return jnp.stack([s2, d2], axis=1)

    idx64 = _pack_idx(64)
    idx128 = _pack_idx(128)

    x_pad = jnp.zeros((N_PAD, D_IN), jnp.float32).at[:N_NODES].set(x)

    xl1, xr1, lin1 = _mm3(x_pad, Wl1, Wr1, L1w)
    zeros1 = jnp.zeros((N_PAD, D_HID), jnp.float32)
    acc1, den1 = _make_edge_kernel(D_HID)(xl1, xr1, att1, idx64, zeros1)

    xl2, xr2, lin2 = _mid(acc1, den1, lin1, b1.reshape(1, -1),
                          L1b.reshape(1, -1), Wl2, Wr2, L2w)
    zeros2 = jnp.zeros((N_PAD, D_OUT), jnp.float32)
    acc2, den2 = _make_edge_kernel(D_OUT)(xl2, xr2, att2, idx128, zeros2)

    out = _final(acc2, den2, lin2, b2.reshape(1, -1), L2b.reshape(1, -1))
    return (out[:N_NODES], edge_index)


# speedup vs baseline: 1.0877x; 1.0877x over previous
import functools

import jax
import jax.numpy as jnp
from jax import lax
from jax.experimental import pallas as pl
from jax.experimental.pallas import tpu as pltpu
from jax.experimental.pallas import tpu_sc as plsc

N_NODES = 10000
N_EDGES = 320000
D_IN = 128
D_HID = 128
D_OUT = 64

N_PAD = 10240
NUM_TILES = 32
E_PER_TILE = 10240
E_PAD = E_PER_TILE * NUM_TILES


def _allreduce_sum16(v):
    lane = lax.iota(jnp.int32, 16)
    for sh in (8, 4, 2, 1):
        v = v + jnp.take_along_axis(v, jnp.bitwise_xor(lane, sh), axis=0)
    return v


def _make_edge_kernel(d, interpret=False):
    chunk = 64 if d >= 128 else 128
    k_sub = d // 16
    rows_per_tile = N_PAD // 16
    n_chunks = E_PER_TILE // chunk
    mesh = plsc.VectorSubcoreMesh(
        core_axis_name="c", subcore_axis_name="s", num_cores=2, num_subcores=16
    )

    @functools.partial(
        pl.kernel,
        out_type=(jax.ShapeDtypeStruct((2, N_PAD, d), jnp.float32),
                  jax.ShapeDtypeStruct((NUM_TILES, N_PAD), jnp.float32)),
        mesh=mesh,
        scratch_types=[
            pltpu.VMEM((2, chunk), jnp.int32),
            pltpu.VMEM((2, chunk), jnp.int32),
            pltpu.VMEM((chunk, d), jnp.float32),
            pltpu.VMEM((chunk, d), jnp.float32),
            pltpu.VMEM((chunk, d), jnp.float32),
            pltpu.VMEM((chunk, d), jnp.float32),
            pltpu.VMEM((d,), jnp.float32),
            pltpu.VMEM((N_PAD,), jnp.float32),
            pltpu.VMEM((128,), jnp.float32),
            pltpu.VMEM_SHARED((N_PAD, d), jnp.float32),
            pltpu.SemaphoreType.DMA,
            pltpu.SemaphoreType.DMA,
        ],
        compiler_params=pltpu.CompilerParams(use_tc_tiling_on_sc=False,
                                             needs_layout_passes=False),
        interpret=interpret,
    )
    def edge_kernel(xl_hbm, xr_hbm, att_hbm, idx_hbm, zeros_hbm,
                    out_hbm, den_hbm, sidx0, sidx1, arows0,
                    brows0, arows1, brows1, attv, denv, exbuf, acc_sp,
                    sem0, sem1):
        cid = lax.axis_index("c")
        sid = lax.axis_index("s")
        wid = sid * 2 + cid
        row0 = sid * rows_per_tile

        pltpu.sync_copy(zeros_hbm.at[pl.ds(row0, rows_per_tile)],
                        acc_sp.at[pl.ds(row0, rows_per_tile)])
        pltpu.sync_copy(att_hbm, attv)
        zv = jnp.zeros((16,), jnp.float32)

        def zero_body(i, carry):
            denv[pl.ds(i * 16, 16)] = zv
            return carry

        lax.fori_loop(0, N_PAD // 16, zero_body, 0)
        plsc.subcore_barrier()

        att_vs = [attv[pl.ds(k * 16, 16)] for k in range(k_sub)]
        lane = lax.iota(jnp.int32, 16)
        lowbit = jnp.bitwise_and(lane, 1)
        oddlane = lowbit == 1
        lane01 = lane < 2
        cbase = wid * n_chunks
        bufs = [(sidx0, arows0, brows0, sem0),
                (sidx1, arows1, brows1, sem1)]

        def start_gathers(ci, buf):
            sidx, arows, brows, sem = buf
            pltpu.sync_copy(idx_hbm.at[cbase + ci], sidx)
            pltpu.async_copy(xl_hbm.at[sidx.at[0]], arows, sem)
            pltpu.async_copy(xr_hbm.at[sidx.at[1]], brows, sem)

        def wait_gathers(buf):
            sidx, arows, brows, sem = buf
            pltpu.make_async_copy(xl_hbm.at[sidx.at[0]], arows, sem).wait()
            pltpu.make_async_copy(xr_hbm.at[sidx.at[1]], brows, sem).wait()

        start_gathers(0, bufs[0])

        def outer_body(co, carry):
            for b in (0, 1):
                ci = 2 * co + b
                sidx, arows, brows, sem = bufs[b]
                dstv = sidx.at[1]
                wait_gathers(bufs[b])
                start_gathers(jnp.minimum(ci + 1, n_chunks - 1), bufs[1 - b])

                @plsc.parallel_loop(0, chunk // 2, unroll=2)
                def edge_loop(jj):
                    j0 = 2 * jj
                    j1 = j0 + 1
                    avs0, avs1 = [], []
                    s0 = jnp.zeros((16,), jnp.float32)
                    s1 = jnp.zeros((16,), jnp.float32)
                    for k in range(k_sub):
                        a0 = arows[j0, pl.ds(k * 16, 16)]
                        b0 = brows[j0, pl.ds(k * 16, 16)]
                        a1 = arows[j1, pl.ds(k * 16, 16)]
                        b1 = brows[j1, pl.ds(k * 16, 16)]
                        z0 = a0 + b0
                        z1 = a1 + b1
                        s0 = s0 + jnp.maximum(z0, 0.2 * z0) * att_vs[k]
                        s1 = s1 + jnp.maximum(z1, 0.2 * z1) * att_vs[k]
                        avs0.append(a0)
                        avs1.append(a1)
                    ex0 = jnp.exp(_allreduce_sum16(s0))
                    ex1 = jnp.exp(_allreduce_sum16(s1))
                    for k in range(k_sub):
                        arows[j0, pl.ds(k * 16, 16)] = avs0[k] * ex0
                        arows[j1, pl.ds(k * 16, 16)] = avs1[k] * ex1
                    exmine = jnp.where(oddlane, ex1, ex0)
                    plsc.store_scatter(exbuf, [j0 + lowbit], exmine,
                                       mask=lane01)

                def den_sweep(g, carry2):
                    djs = sidx[1, pl.ds(g * 16, 16)]
                    exs = exbuf[pl.ds(g * 16, 16)]
                    old = plsc.load_gather(denv, [djs])
                    tot = exs
                    for sh in range(1, 16):
                        perm = jnp.bitwise_xor(lane, sh)
                        dperm = jnp.take_along_axis(djs, perm, axis=0)
                        eperm = jnp.take_along_axis(exs, perm, axis=0)
                        tot = tot + jnp.where(djs == dperm, eperm, 0.0)
                    plsc.store_scatter(denv, [djs], old + tot)
                    return carry2

                lax.fori_loop(0, chunk // 16, den_sweep, 0)
                pltpu.sync_copy(arows, acc_sp.at[dstv], add=True)
            return carry

        lax.fori_loop(0, n_chunks // 2, outer_body, 0)
        wait_gathers(bufs[0])
        pltpu.sync_copy(denv, den_hbm.at[wid])
        plsc.subcore_barrier()
        pltpu.sync_copy(acc_sp.at[pl.ds(row0, rows_per_tile)],
                        out_hbm.at[cid, pl.ds(row0, rows_per_tile)])

    return edge_kernel


_ROWS = 640
_GRID = N_PAD // _ROWS


def _mm3(x, w1, w2, w3, interpret=False):
    dm = w1.shape[1]

    def body(x_ref, w1_ref, w2_ref, w3_ref, o1_ref, o2_ref, o3_ref):
        xb = x_ref[...]
        o1_ref[...] = jnp.dot(xb, w1_ref[...], preferred_element_type=jnp.float32)
        o2_ref[...] = jnp.dot(xb, w2_ref[...], preferred_element_type=jnp.float32)
        o3_ref[...] = jnp.dot(xb, w3_ref[...], preferred_element_type=jnp.float32)

    k = x.shape[1]
    return pl.pallas_call(
        body,
        grid=(_GRID,),
        in_specs=[
            pl.BlockSpec((_ROWS, k), lambda i: (i, 0)),
            pl.BlockSpec((k, dm), lambda i: (0, 0)),
            pl.BlockSpec((k, dm), lambda i: (0, 0)),
            pl.BlockSpec((k, dm), lambda i: (0, 0)),
        ],
        out_specs=[pl.BlockSpec((_ROWS, dm), lambda i: (i, 0))] * 3,
        out_shape=[jax.ShapeDtypeStruct((N_PAD, dm), jnp.float32)] * 3,
        interpret=interpret,
    )(x, w1, w2, w3)


def _mid(acc, den, lin1, b1, l1b, wl2, wr2, l2w, interpret=False):
    d = lin1.shape[1]
    dm = wl2.shape[1]

    def body(acc_ref, den_ref, lin_ref, b1_ref, l1b_ref, wl_ref, wr_ref,
             lw_ref, o1_ref, o2_ref, o3_ref):
        num = acc_ref[0] + acc_ref[1]
        den_b = jnp.sum(den_ref[...], axis=0)[:, None]
        h = num / (den_b + 1e-16) + lin_ref[...] + b1_ref[...] + l1b_ref[...]
        h = jnp.maximum(h, 0.0)
        o1_ref[...] = jnp.dot(h, wl_ref[...], preferred_element_type=jnp.float32)
        o2_ref[...] = jnp.dot(h, wr_ref[...], preferred_element_type=jnp.float32)
        o3_ref[...] = jnp.dot(h, lw_ref[...], preferred_element_type=jnp.float32)

    return pl.pallas_call(
        body,
        grid=(_GRID,),
        in_specs=[
            pl.BlockSpec((2, _ROWS, d), lambda i: (0, i, 0)),
            pl.BlockSpec((NUM_TILES, _ROWS), lambda i: (0, i)),
            pl.BlockSpec((_ROWS, d), lambda i: (i, 0)),
            pl.BlockSpec((1, d), lambda i: (0, 0)),
            pl.BlockSpec((1, d), lambda i: (0, 0)),
            pl.BlockSpec((d, dm), lambda i: (0, 0)),
            pl.BlockSpec((d, dm), lambda i: (0, 0)),
            pl.BlockSpec((d, dm), lambda i: (0, 0)),
        ],
        out_specs=[pl.BlockSpec((_ROWS, dm), lambda i: (i, 0))] * 3,
        out_shape=[jax.ShapeDtypeStruct((N_PAD, dm), jnp.float32)] * 3,
        interpret=interpret,
    )(acc, den, lin1, b1, l1b, wl2, wr2, l2w)


def _final(acc, den, lin2, b2, l2b, interpret=False):
    d = lin2.shape[1]

    def body(acc_ref, den_ref, lin_ref, b2_ref, l2b_ref, o_ref):
        num = acc_ref[0] + acc_ref[1]
        den_b = jnp.sum(den_ref[...], axis=0)[:, None]
        o = num / (den_b + 1e-16) + lin_ref[...] + b2_ref[...] + l2b_ref[...]
        m = jnp.max(o, axis=1, keepdims=True)
        ex = jnp.exp(o - m)
        lse = m + jnp.log(jnp.sum(ex, axis=1, keepdims=True))
        o_ref[...] = o - lse

    return pl.pallas_call(
        body,
        grid=(_GRID,),
        in_specs=[
            pl.BlockSpec((2, _ROWS, d), lambda i: (0, i, 0)),
            pl.BlockSpec((NUM_TILES, _ROWS), lambda i: (0, i)),
            pl.BlockSpec((_ROWS, d), lambda i: (i, 0)),
            pl.BlockSpec((1, d), lambda i: (0, 0)),
            pl.BlockSpec((1, d), lambda i: (0, 0)),
        ],
        out_specs=pl.BlockSpec((_ROWS, d), lambda i: (i, 0)),
        out_shape=jax.ShapeDtypeStruct((N_PAD, d), jnp.float32),
        interpret=interpret,
    )(acc, den, lin2, b2, l2b)


def kernel(x, edge_index, Wl1, Wr1, att1, b1, L1w, L1b, Wl2, Wr2, att2, b2,
           L2w, L2b):
    src = edge_index[0].astype(jnp.int32)
    dst = edge_index[1].astype(jnp.int32)
    per_tile_pad = E_PER_TILE - N_EDGES // NUM_TILES
    pad_rows = N_NODES + jnp.arange(per_tile_pad, dtype=jnp.int32)
    pad_blk = jnp.broadcast_to(pad_rows, (NUM_TILES, per_tile_pad))

    def _pad_edges(v):
        v = v.reshape(NUM_TILES, N_EDGES // NUM_TILES)
        return jnp.concatenate([v, pad_blk], axis=1).reshape(-1)

    src_p = _pad_edges(src)
    dst_p = _pad_edges(dst)

    def _pack_idx(chunk):
        s2 = src_p.reshape(-1, chunk)
        d2 = dst_p.reshape(-1, chunk)
        return jnp.stack([s2, d2], axis=1)

    idx64 = _pack_idx(64)
    idx128 = _pack_idx(128)

    x_pad = jnp.zeros((N_PAD, D_IN), jnp.float32).at[:N_NODES].set(x)

    xl1, xr1, lin1 = _mm3(x_pad, Wl1, Wr1, L1w)
    zeros1 = jnp.zeros((N_PAD, D_HID), jnp.float32)
    acc1, den1 = _make_edge_kernel(D_HID)(xl1, xr1, att1, idx64, zeros1)

    xl2, xr2, lin2 = _mid(acc1, den1, lin1, b1.reshape(1, -1),
                          L1b.reshape(1, -1), Wl2, Wr2, L2w)
    zeros2 = jnp.zeros((N_PAD, D_OUT), jnp.float32)
    acc2, den2 = _make_edge_kernel(D_OUT)(xl2, xr2, att2, idx128, zeros2)

    out = _final(acc2, den2, lin2, b2.reshape(1, -1), L2b.reshape(1, -1))
    return (out[:N_NODES], edge_index)


# speedup vs baseline: 1.0879x; 1.0002x over previous
import functools

import jax
import jax.numpy as jnp
from jax import lax
from jax.experimental import pallas as pl
from jax.experimental.pallas import tpu as pltpu
from jax.experimental.pallas import tpu_sc as plsc

N_NODES = 10000
N_EDGES = 320000
D_IN = 128
D_HID = 128
D_OUT = 64

N_PAD = 10240
NUM_TILES = 32
E_PER_TILE = 10240
E_PAD = E_PER_TILE * NUM_TILES


def _allreduce_sum16(v):
    lane = lax.iota(jnp.int32, 16)
    for sh in (8, 4, 2, 1):
        v = v + jnp.take_along_axis(v, jnp.bitwise_xor(lane, sh), axis=0)
    return v


def _make_edge_kernel(d):
    chunk = 64 if d >= 128 else 128
    k_sub = d // 16
    rows_per_tile = N_PAD // 16
    n_chunks = E_PER_TILE // chunk
    mesh = plsc.VectorSubcoreMesh(
        core_axis_name="c", subcore_axis_name="s", num_cores=2, num_subcores=16
    )

    @functools.partial(
        pl.kernel,
        out_type=(jax.ShapeDtypeStruct((2, N_PAD, d), jnp.float32),
                  jax.ShapeDtypeStruct((NUM_TILES, N_PAD), jnp.float32)),
        mesh=mesh,
        scratch_types=[
            pltpu.VMEM((2, chunk), jnp.int32),
            pltpu.VMEM((2, chunk), jnp.int32),
            pltpu.VMEM((chunk, d), jnp.float32),
            pltpu.VMEM((chunk, d), jnp.float32),
            pltpu.VMEM((chunk, d), jnp.float32),
            pltpu.VMEM((chunk, d), jnp.float32),
            pltpu.VMEM((d,), jnp.float32),
            pltpu.VMEM((N_PAD,), jnp.float32),
            pltpu.VMEM((128,), jnp.float32),
            pltpu.VMEM_SHARED((N_PAD, d), jnp.float32),
            pltpu.SemaphoreType.DMA,
            pltpu.SemaphoreType.DMA,
        ],
        compiler_params=pltpu.CompilerParams(use_tc_tiling_on_sc=False,
                                             needs_layout_passes=False),
    )
    def edge_kernel(xl_hbm, xr_hbm, att_hbm, idx_hbm, zeros_hbm,
                    out_hbm, den_hbm, sidx0, sidx1, arows0,
                    brows0, arows1, brows1, attv, denv, exbuf, acc_sp,
                    sem0, sem1):
        cid = lax.axis_index("c")
        sid = lax.axis_index("s")
        wid = sid * 2 + cid
        row0 = sid * rows_per_tile

        pltpu.sync_copy(zeros_hbm.at[pl.ds(row0, rows_per_tile)],
                        acc_sp.at[pl.ds(row0, rows_per_tile)])
        pltpu.sync_copy(att_hbm, attv)
        zv = jnp.zeros((16,), jnp.float32)

        def zero_body(i, carry):
            denv[pl.ds(i * 16, 16)] = zv
            return carry

        lax.fori_loop(0, N_PAD // 16, zero_body, 0)
        plsc.subcore_barrier()

        att_vs = [attv[pl.ds(k * 16, 16)] for k in range(k_sub)]
        lane = lax.iota(jnp.int32, 16)
        lowbit = jnp.bitwise_and(lane, 1)
        oddlane = lowbit == 1
        lane01 = lane < 2
        cbase = wid * n_chunks
        bufs = [(sidx0, arows0, brows0, sem0),
                (sidx1, arows1, brows1, sem1)]

        def start_gathers(ci, buf):
            sidx, arows, brows, sem = buf
            pltpu.sync_copy(idx_hbm.at[cbase + ci], sidx)
            pltpu.async_copy(xl_hbm.at[sidx.at[0]], arows, sem)
            pltpu.async_copy(xr_hbm.at[sidx.at[1]], brows, sem)

        def wait_gathers(buf):
            sidx, arows, brows, sem = buf
            pltpu.make_async_copy(xl_hbm.at[sidx.at[0]], arows, sem).wait()
            pltpu.make_async_copy(xr_hbm.at[sidx.at[1]], brows, sem).wait()

        start_gathers(0, bufs[0])

        def outer_body(co, carry):
            for b in (0, 1):
                ci = 2 * co + b
                sidx, arows, brows, sem = bufs[b]
                dstv = sidx.at[1]
                wait_gathers(bufs[b])
                start_gathers(jnp.minimum(ci + 1, n_chunks - 1), bufs[1 - b])

                @plsc.parallel_loop(0, chunk // 2, unroll=2)
                def edge_loop(jj):
                    j0 = 2 * jj
                    j1 = j0 + 1
                    avs0, avs1 = [], []
                    s0 = jnp.zeros((16,), jnp.float32)
                    s1 = jnp.zeros((16,), jnp.float32)
                    for k in range(k_sub):
                        a0 = arows[j0, pl.ds(k * 16, 16)]
                        b0 = brows[j0, pl.ds(k * 16, 16)]
                        a1 = arows[j1, pl.ds(k * 16, 16)]
                        b1 = brows[j1, pl.ds(k * 16, 16)]
                        z0 = a0 + b0
                        z1 = a1 + b1
                        s0 = s0 + jnp.maximum(z0, 0.2 * z0) * att_vs[k]
                        s1 = s1 + jnp.maximum(z1, 0.2 * z1) * att_vs[k]
                        avs0.append(a0)
                        avs1.append(a1)
                    ex0 = jnp.exp(_allreduce_sum16(s0))
                    ex1 = jnp.exp(_allreduce_sum16(s1))
                    for k in range(k_sub):
                        arows[j0, pl.ds(k * 16, 16)] = avs0[k] * ex0
                        arows[j1, pl.ds(k * 16, 16)] = avs1[k] * ex1
                    exmine = jnp.where(oddlane, ex1, ex0)
                    plsc.store_scatter(exbuf, [j0 + lowbit], exmine,
                                       mask=lane01)

                def den_sweep(g, carry2):
                    djs = sidx[1, pl.ds(g * 16, 16)]
                    exs = exbuf[pl.ds(g * 16, 16)]
                    old = plsc.load_gather(denv, [djs])
                    tot = exs
                    for sh in range(1, 16):
                        perm = jnp.bitwise_xor(lane, sh)
                        dperm = jnp.take_along_axis(djs, perm, axis=0)
                        eperm = jnp.take_along_axis(exs, perm, axis=0)
                        tot = tot + jnp.where(djs == dperm, eperm, 0.0)
                    plsc.store_scatter(denv, [djs], old + tot)
                    return carry2

                lax.fori_loop(0, chunk // 16, den_sweep, 0)
                pltpu.sync_copy(arows, acc_sp.at[dstv], add=True)
            return carry

        lax.fori_loop(0, n_chunks // 2, outer_body, 0)
        wait_gathers(bufs[0])
        pltpu.sync_copy(denv, den_hbm.at[wid])
        plsc.subcore_barrier()
        pltpu.sync_copy(acc_sp.at[pl.ds(row0, rows_per_tile)],
                        out_hbm.at[cid, pl.ds(row0, rows_per_tile)])

    return edge_kernel


_ROWS = 640
_GRID = N_PAD // _ROWS


def _mm3(x, w1, w2, w3):
    dm = w1.shape[1]

    def body(x_ref, w1_ref, w2_ref, w3_ref, o1_ref, o2_ref, o3_ref):
        xb = x_ref[...]
        o1_ref[...] = jnp.dot(xb, w1_ref[...], preferred_element_type=jnp.float32)
        o2_ref[...] = jnp.dot(xb, w2_ref[...], preferred_element_type=jnp.float32)
        o3_ref[...] = jnp.dot(xb, w3_ref[...], preferred_element_type=jnp.float32)

    k = x.shape[1]
    return pl.pallas_call(
        body,
        grid=(_GRID,),
        in_specs=[
            pl.BlockSpec((_ROWS, k), lambda i: (i, 0)),
            pl.BlockSpec((k, dm), lambda i: (0, 0)),
            pl.BlockSpec((k, dm), lambda i: (0, 0)),
            pl.BlockSpec((k, dm), lambda i: (0, 0)),
        ],
        out_specs=[pl.BlockSpec((_ROWS, dm), lambda i: (i, 0))] * 3,
        out_shape=[jax.ShapeDtypeStruct((N_PAD, dm), jnp.float32)] * 3,
    )(x, w1, w2, w3)


def _mid(acc, den, lin1, b1, l1b, wl2, wr2, l2w):
    d = lin1.shape[1]
    dm = wl2.shape[1]

    def body(acc_ref, den_ref, lin_ref, b1_ref, l1b_ref, wl_ref, wr_ref,
             lw_ref, o1_ref, o2_ref, o3_ref):
        num = acc_ref[0] + acc_ref[1]
        den_b = jnp.sum(den_ref[...], axis=0)[:, None]
        h = num / (den_b + 1e-16) + lin_ref[...] + b1_ref[...] + l1b_ref[...]
        h = jnp.maximum(h, 0.0)
        o1_ref[...] = jnp.dot(h, wl_ref[...], preferred_element_type=jnp.float32)
        o2_ref[...] = jnp.dot(h, wr_ref[...], preferred_element_type=jnp.float32)
        o3_ref[...] = jnp.dot(h, lw_ref[...], preferred_element_type=jnp.float32)

    return pl.pallas_call(
        body,
        grid=(_GRID,),
        in_specs=[
            pl.BlockSpec((2, _ROWS, d), lambda i: (0, i, 0)),
            pl.BlockSpec((NUM_TILES, _ROWS), lambda i: (0, i)),
            pl.BlockSpec((_ROWS, d), lambda i: (i, 0)),
            pl.BlockSpec((1, d), lambda i: (0, 0)),
            pl.BlockSpec((1, d), lambda i: (0, 0)),
            pl.BlockSpec((d, dm), lambda i: (0, 0)),
            pl.BlockSpec((d, dm), lambda i: (0, 0)),
            pl.BlockSpec((d, dm), lambda i: (0, 0)),
        ],
        out_specs=[pl.BlockSpec((_ROWS, dm), lambda i: (i, 0))] * 3,
        out_shape=[jax.ShapeDtypeStruct((N_PAD, dm), jnp.float32)] * 3,
    )(acc, den, lin1, b1, l1b, wl2, wr2, l2w)


def _final(acc, den, lin2, b2, l2b):
    d = lin2.shape[1]

    def body(acc_ref, den_ref, lin_ref, b2_ref, l2b_ref, o_ref):
        num = acc_ref[0] + acc_ref[1]
        den_b = jnp.sum(den_ref[...], axis=0)[:, None]
        o = num / (den_b + 1e-16) + lin_ref[...] + b2_ref[...] + l2b_ref[...]
        m = jnp.max(o, axis=1, keepdims=True)
        ex = jnp.exp(o - m)
        lse = m + jnp.log(jnp.sum(ex, axis=1, keepdims=True))
        o_ref[...] = o - lse

    return pl.pallas_call(
        body,
        grid=(_GRID,),
        in_specs=[
            pl.BlockSpec((2, _ROWS, d), lambda i: (0, i, 0)),
            pl.BlockSpec((NUM_TILES, _ROWS), lambda i: (0, i)),
            pl.BlockSpec((_ROWS, d), lambda i: (i, 0)),
            pl.BlockSpec((1, d), lambda i: (0, 0)),
            pl.BlockSpec((1, d), lambda i: (0, 0)),
        ],
        out_specs=pl.BlockSpec((_ROWS, d), lambda i: (i, 0)),
        out_shape=jax.ShapeDtypeStruct((N_PAD, d), jnp.float32),
    )(acc, den, lin2, b2, l2b)


def kernel(x, edge_index, Wl1, Wr1, att1, b1, L1w, L1b, Wl2, Wr2, att2, b2,
           L2w, L2b):
    src = edge_index[0].astype(jnp.int32)
    dst = edge_index[1].astype(jnp.int32)
    per_tile_pad = E_PER_TILE - N_EDGES // NUM_TILES
    pad_rows = N_NODES + jnp.arange(per_tile_pad, dtype=jnp.int32)
    pad_blk = jnp.broadcast_to(pad_rows, (NUM_TILES, per_tile_pad))

    def _pad_edges(v):
        v = v.reshape(NUM_TILES, N_EDGES // NUM_TILES)
        return jnp.concatenate([v, pad_blk], axis=1).reshape(-1)

    src_p = _pad_edges(src)
    dst_p = _pad_edges(dst)

    def _pack_idx(chunk):
        s2 = src_p.reshape(-1, chunk)
        d2 = dst_p.reshape(-1, chunk)
        return jnp.stack([s2, d2], axis=1)

    idx64 = _pack_idx(64)
    idx128 = _pack_idx(128)

    x_pad = jnp.zeros((N_PAD, D_IN), jnp.float32).at[:N_NODES].set(x)

    xl1, xr1, lin1 = _mm3(x_pad, Wl1, Wr1, L1w)
    zeros1 = jnp.zeros((N_PAD, D_HID), jnp.float32)
    acc1, den1 = _make_edge_kernel(D_HID)(xl1, xr1, att1, idx64, zeros1)

    xl2, xr2, lin2 = _mid(acc1, den1, lin1, b1.reshape(1, -1),
                          L1b.reshape(1, -1), Wl2, Wr2, L2w)
    zeros2 = jnp.zeros((N_PAD, D_OUT), jnp.float32)
    acc2, den2 = _make_edge_kernel(D_OUT)(xl2, xr2, att2, idx128, zeros2)

    out = _final(acc2, den2, lin2, b2.reshape(1, -1), L2b.reshape(1, -1))
    return (out[:N_NODES], edge_index)
